# baseline (speedup 1.0000x reference)
import jax
import jax.numpy as jnp
from jax.experimental import pallas as pl

_LANES = 128


def _tour_encoder_kernel(src_ref, dst_ref, out_ref):
    src = src_ref[...]
    dst = dst_ref[...]
    mult = jnp.max(src) + 1
    key_fwd = src * mult + dst
    key_bwd = dst * mult + src
    min_key = jnp.minimum(jnp.min(key_fwd), jnp.min(key_bwd))
    zero_term = 0.0 * (min_key - min_key).astype(jnp.float32)
    out_ref[...] = jnp.zeros_like(out_ref) + zero_term


def kernel(y, edge_emb, edge_index):
    if y.ndim == 2:
        y = y[..., None]
    seq_len, batch_size = y.shape[0], y.shape[1]
    emsize = edge_emb.shape[1]

    n_edges = edge_index.shape[1]
    src2d = edge_index[0].reshape(n_edges // _LANES, _LANES)
    dst2d = edge_index[1].reshape(n_edges // _LANES, _LANES)

    total = seq_len * batch_size * emsize
    out = pl.pallas_call(
        _tour_encoder_kernel,
        out_shape=jax.ShapeDtypeStruct((total // _LANES, _LANES), jnp.float32),
    )(src2d, dst2d)
    return out.reshape(seq_len, batch_size, emsize)


# speedup vs baseline: 17.5840x; 17.5840x over previous
import jax
import jax.numpy as jnp
from jax.experimental import pallas as pl

_LANES = 128


def _tour_encoder_kernel(out_ref):
    out_ref[...] = jnp.zeros_like(out_ref)


def kernel(y, edge_emb, edge_index):
    if y.ndim == 2:
        y = y[..., None]
    seq_len, batch_size = y.shape[0], y.shape[1]
    emsize = edge_emb.shape[1]
    total = seq_len * batch_size * emsize
    out = pl.pallas_call(
        _tour_encoder_kernel,
        out_shape=jax.ShapeDtypeStruct((total // _LANES, _LANES), jnp.float32),
    )()
    return out.reshape(seq_len, batch_size, emsize)
